# Initial kernel scaffold; baseline (speedup 1.0000x reference)
import jax
import jax.numpy as jnp
from jax import lax
from jax.experimental import pallas as pl
from jax.experimental.pallas import tpu as pltpu
from jax.experimental.pallas import tpu_sc as plsc

N = 10000
E = 160000
D = 128
NT = 16
ET = E // NT
C = 80
NCH = ET // C
RPT = N // NT
NG = C // 16


def _run_relation(s, f, src_h, d2_h, d1_h, a_h, out_h,
                  src_v, dst2d_v, dst1d_v, rows_s, rows_d, ex_v, avmem, denl,
                  acc, den):
    base = s * ET
    pltpu.sync_copy(src_h.at[pl.ds(base, ET)], src_v)
    pltpu.sync_copy(d2_h.at[pl.ds(s * NCH, NCH)], dst2d_v)
    pltpu.sync_copy(d1_h.at[pl.ds(base, ET)], dst1d_v)
    pltpu.sync_copy(a_h, avmem)
    a_regs = [avmem[pl.ds(j * 16, 16)] for j in range(8)]

    def chunk1(i, _):
        off = i * C
        pltpu.sync_copy(f.at[src_v.at[pl.ds(off, C)]], rows_s)
        pltpu.sync_copy(f.at[dst1d_v.at[pl.ds(off, C)]], rows_d)

        def edge(e, _):
            acc_v = jnp.zeros((16,), jnp.float32)
            for j in range(8):
                sv = rows_s[e, pl.ds(j * 16, 16)]
                dv = rows_d[e, pl.ds(j * 16, 16)]
                t = sv + dv
                lr = jnp.maximum(t, 0.2 * t)
                acc_v = acc_v + lr * a_regs[j]
            ex_v[off + e] = jnp.sum(acc_v)
            return 0
        lax.fori_loop(0, C, edge, 0)

        def grp(g, _):
            p = off + g * 16
            ex_v[pl.ds(p, 16)] = jnp.exp(ex_v[pl.ds(p, 16)])
            return 0
        lax.fori_loop(0, NG, grp, 0)
        pltpu.sync_copy(ex_v.at[pl.ds(off, C)], den.at[dst2d_v.at[i]], add=True)
        return 0
    lax.fori_loop(0, NCH, chunk1, 0)

    plsc.subcore_barrier()
    pltpu.sync_copy(den, denl)

    def chunk2(i, _):
        off = i * C
        pltpu.sync_copy(f.at[src_v.at[pl.ds(off, C)]], rows_s)

        def grp(g, _):
            p = off + g * 16
            dst16 = dst1d_v[pl.ds(p, 16)]
            den16 = plsc.load_gather(denl, [dst16])
            ex16 = ex_v[pl.ds(p, 16)]
            al16 = ex16 / jnp.maximum(den16, 1e-9)
            for e in range(16):
                a_s = al16[e]
                ee = g * 16 + e
                for j in range(8):
                    rows_s[ee, pl.ds(j * 16, 16)] = (
                        rows_s[ee, pl.ds(j * 16, 16)] * a_s)
            return 0
        lax.fori_loop(0, NG, grp, 0)
        pltpu.sync_copy(rows_s, acc.at[dst2d_v.at[i]], add=True)
        return 0
    lax.fori_loop(0, NCH, chunk2, 0)

    plsc.subcore_barrier()
    pltpu.sync_copy(acc.at[pl.ds(s * RPT, RPT)], out_h.at[pl.ds(s * RPT, RPT)])


def _sc_body(f0, f1, src0, src1, d20, d21, d10, d11, a0, a1, znd, zn,
             out0, out1,
             src_v, dst2d_v, dst1d_v, rows_s, rows_d, ex_v, avmem, denl,
             acc, den):
    c = lax.axis_index("c")
    s = lax.axis_index("s")
    pltpu.sync_copy(znd.at[pl.ds(s * RPT, RPT)], acc.at[pl.ds(s * RPT, RPT)])

    @pl.when(s < 10)
    def _():
        pltpu.sync_copy(zn.at[pl.ds(s * 1000, 1000)],
                        den.at[pl.ds(s * 1000, 1000)])
    plsc.subcore_barrier()

    @pl.when(c == 0)
    def _():
        _run_relation(s, f0, src0, d20, d10, a0, out0,
                      src_v, dst2d_v, dst1d_v, rows_s, rows_d, ex_v, avmem,
                      denl, acc, den)

    @pl.when(c == 1)
    def _():
        _run_relation(s, f1, src1, d21, d11, a1, out1,
                      src_v, dst2d_v, dst1d_v, rows_s, rows_d, ex_v, avmem,
                      denl, acc, den)


_sc_gat = pl.kernel(
    _sc_body,
    out_type=[jax.ShapeDtypeStruct((N, D), jnp.float32),
              jax.ShapeDtypeStruct((N, D), jnp.float32)],
    mesh=plsc.VectorSubcoreMesh(core_axis_name="c", subcore_axis_name="s"),
    scratch_types=[
        pltpu.VMEM((ET,), jnp.int32),
        pltpu.VMEM((NCH, C), jnp.int32),
        pltpu.VMEM((ET,), jnp.int32),
        pltpu.VMEM((C, D), jnp.float32),
        pltpu.VMEM((C, D), jnp.float32),
        pltpu.VMEM((ET,), jnp.float32),
        pltpu.VMEM((D,), jnp.float32),
        pltpu.VMEM((N,), jnp.float32),
        pltpu.VMEM_SHARED((N, D), jnp.float32),
        pltpu.VMEM_SHARED((N,), jnp.float32),
    ],
)


def _tc_pre_body(x_ref, w0_ref, b0_ref, w1_ref, b1_ref, f0_ref, f1_ref):
    x = x_ref[...]
    f0_ref[...] = (jnp.dot(x, w0_ref[...], preferred_element_type=jnp.float32)
                   + b0_ref[...])
    f1_ref[...] = (jnp.dot(x, w1_ref[...], preferred_element_type=jnp.float32)
                   + b1_ref[...])


_tc_pre = pl.pallas_call(
    _tc_pre_body,
    out_shape=[jax.ShapeDtypeStruct((N, D), jnp.float32),
               jax.ShapeDtypeStruct((N, D), jnp.float32)],
)


def _tc_mid_body(g0_ref, g1_ref, hp_ref, w0_ref, b0_ref, w1_ref, b1_ref,
                 h_ref, f0_ref, f1_ref):
    h = jax.nn.relu(g0_ref[...] + g1_ref[...] + 2.0 * hp_ref[...])
    h_ref[...] = h
    f0_ref[...] = (jnp.dot(h, w0_ref[...], preferred_element_type=jnp.float32)
                   + b0_ref[...])
    f1_ref[...] = (jnp.dot(h, w1_ref[...], preferred_element_type=jnp.float32)
                   + b1_ref[...])


_tc_mid = pl.pallas_call(
    _tc_mid_body,
    out_shape=[jax.ShapeDtypeStruct((N, D), jnp.float32),
               jax.ShapeDtypeStruct((N, D), jnp.float32),
               jax.ShapeDtypeStruct((N, D), jnp.float32)],
)


def _tc_post_body(g0_ref, g1_ref, hp_ref, o_ref):
    o_ref[...] = jax.nn.relu(g0_ref[...] + g1_ref[...] + 2.0 * hp_ref[...])


_tc_post = pl.pallas_call(
    _tc_post_body,
    out_shape=jax.ShapeDtypeStruct((N, D), jnp.float32),
)


def kernel(x, edge_index_r0, edge_index_r1,
           W_l0_r0, b_l0_r0, a_l0_r0, W_l0_r1, b_l0_r1, a_l0_r1,
           W_out_r0, b_out_r0, a_out_r0, W_out_r1, b_out_r1, a_out_r1):
    s0, d0 = edge_index_r0[0], edge_index_r0[1]
    s1, d1 = edge_index_r1[0], edge_index_r1[1]
    d20 = d0.reshape(E // C, C)
    d21 = d1.reshape(E // C, C)
    znd = jnp.zeros((N, D), jnp.float32)
    zn = jnp.zeros((N,), jnp.float32)

    f00, f01 = _tc_pre(x, W_l0_r0, b_l0_r0, W_l0_r1, b_l0_r1)
    g00, g01 = _sc_gat(f00, f01, s0, s1, d20, d21, d0, d1,
                       a_l0_r0, a_l0_r1, znd, zn)
    h, f10, f11 = _tc_mid(g00, g01, x, W_out_r0, b_out_r0, W_out_r1, b_out_r1)
    g10, g11 = _sc_gat(f10, f11, s0, s1, d20, d21, d0, d1,
                       a_out_r0, a_out_r1, znd, zn)
    return _tc_post(g10, g11, h)


# speedup vs baseline: 6.8938x; 6.8938x over previous
import jax
import jax.numpy as jnp
from jax import lax
from jax.experimental import pallas as pl
from jax.experimental.pallas import tpu as pltpu
from jax.experimental.pallas import tpu_sc as plsc

N = 10000
E = 160000
D = 128
NT = 16
ET = E // NT
C = 80
NCH = ET // C
RPT = N // NT
NG = C // 16


def _run_relation(s, f, src_h, d2_h, a_h, out_h,
                  srcbuf, dst2d_v, rows_s, rows_d, ex_v, avmem, denbuf,
                  acc, den):
    base = s * ET
    pltpu.sync_copy(d2_h.at[s], dst2d_v)
    pltpu.sync_copy(a_h, avmem)
    a_regs = [avmem[pl.ds(j * 16, 16)] for j in range(8)]

    lane15 = lax.iota(jnp.int32, 16) == 15

    def chunk1(i, _):
        off = i * C
        pltpu.sync_copy(src_h.at[pl.ds(base + off, C)], srcbuf.at[0])
        pltpu.sync_copy(f.at[srcbuf.at[0]], rows_s)
        pltpu.sync_copy(f.at[dst2d_v.at[i]], rows_d)

        def edge(e, _):
            acc_v = jnp.zeros((16,), jnp.float32)
            for j in range(8):
                sv = rows_s[e, pl.ds(j * 16, 16)]
                dv = rows_d[e, pl.ds(j * 16, 16)]
                t = sv + dv
                lr = jnp.maximum(t, 0.2 * t)
                acc_v = acc_v + lr * a_regs[j]
            cs = plsc.cumsum(acc_v)
            idxv = jnp.full((16,), off + e, jnp.int32)
            plsc.store_scatter(ex_v, [idxv], cs, mask=lane15)
            return 0
        lax.fori_loop(0, C, edge, 0)

        def grp(g, _):
            p = off + g * 16
            ex_v[pl.ds(p, 16)] = jnp.exp(ex_v[pl.ds(p, 16)])
            return 0
        lax.fori_loop(0, NG, grp, 0)
        pltpu.sync_copy(ex_v.at[pl.ds(off, C)], den.at[dst2d_v.at[i]], add=True)
        return 0
    lax.fori_loop(0, NCH, chunk1, 0)

    plsc.subcore_barrier()

    def chunk2(i, _):
        off = i * C
        pltpu.sync_copy(src_h.at[pl.ds(base + off, C)], srcbuf.at[0])
        pltpu.sync_copy(f.at[srcbuf.at[0]], rows_s)
        pltpu.sync_copy(den.at[dst2d_v.at[i]], denbuf)

        def grp(g, _):
            p = off + g * 16
            den16 = denbuf[pl.ds(g * 16, 16)]
            ex16 = ex_v[pl.ds(p, 16)]
            al16 = ex16 / jnp.maximum(den16, 1e-9)
            for e in range(16):
                a_s = al16[e]
                ee = g * 16 + e
                for j in range(8):
                    rows_s[ee, pl.ds(j * 16, 16)] = (
                        rows_s[ee, pl.ds(j * 16, 16)] * a_s)
            return 0
        lax.fori_loop(0, NG, grp, 0)
        pltpu.sync_copy(rows_s, acc.at[dst2d_v.at[i]], add=True)
        return 0
    lax.fori_loop(0, NCH, chunk2, 0)

    plsc.subcore_barrier()

    @pl.when(s < NT - 1)
    def _():
        base = s * 624
        for k in range(7):
            pltpu.sync_copy(acc.at[pl.ds(base + k * 80, 80)], rows_s)
            pltpu.sync_copy(rows_s, out_h.at[pl.ds(base + k * 80, 80)])
        pltpu.sync_copy(acc.at[pl.ds(base + 560, 64)], rows_s.at[pl.ds(0, 64)])
        pltpu.sync_copy(rows_s.at[pl.ds(0, 64)], out_h.at[pl.ds(base + 560, 64)])

    @pl.when(s == NT - 1)
    def _():
        for k in range(8):
            pltpu.sync_copy(acc.at[pl.ds(9360 + k * 80, 80)], rows_s)
            pltpu.sync_copy(rows_s, out_h.at[pl.ds(9360 + k * 80, 80)])


def _sc_body(f0, f1, src0, src1, d20, d21, a0, a1,
             out0, out1,
             srcbuf, dst2d_v, rows_s, rows_d, ex_v, avmem, denbuf,
             acc, den):
    c = lax.axis_index("c")
    s = lax.axis_index("s")

    z16 = jnp.zeros((16,), jnp.float32)

    def zrow(r, _):
        for j in range(8):
            rows_s[r, pl.ds(j * 16, 16)] = z16
        return 0
    lax.fori_loop(0, C, zrow, 0)

    def zex(k, _):
        ex_v[pl.ds(k * 16, 16)] = z16
        return 0
    lax.fori_loop(0, 40, zex, 0)

    @pl.when(s < NT - 1)
    def _():
        base = s * 624
        for k in range(7):
            pltpu.sync_copy(rows_s, acc.at[pl.ds(base + k * 80, 80)])
        pltpu.sync_copy(rows_s.at[pl.ds(0, 64)], acc.at[pl.ds(base + 560, 64)])
        pltpu.sync_copy(ex_v.at[pl.ds(0, 640)], den.at[pl.ds(s * 640, 640)])

    @pl.when(s == NT - 1)
    def _():
        for k in range(8):
            pltpu.sync_copy(rows_s, acc.at[pl.ds(9360 + k * 80, 80)])
        pltpu.sync_copy(ex_v.at[pl.ds(0, 400)], den.at[pl.ds(9600, 400)])
    plsc.subcore_barrier()

    @pl.when(c == 0)
    def _():
        _run_relation(s, f0, src0, d20, a0, out0,
                      srcbuf, dst2d_v, rows_s, rows_d, ex_v, avmem,
                      denbuf, acc, den)

    @pl.when(c == 1)
    def _():
        _run_relation(s, f1, src1, d21, a1, out1,
                      srcbuf, dst2d_v, rows_s, rows_d, ex_v, avmem,
                      denbuf, acc, den)


_sc_gat = pl.kernel(
    _sc_body,
    out_type=[jax.ShapeDtypeStruct((N, D), jnp.float32),
              jax.ShapeDtypeStruct((N, D), jnp.float32)],
    mesh=plsc.VectorSubcoreMesh(core_axis_name="c", subcore_axis_name="s"),
    compiler_params=pltpu.CompilerParams(needs_layout_passes=False),
    scratch_types=[
        pltpu.VMEM((1, C), jnp.int32),
        pltpu.VMEM((NCH, C), jnp.int32),
        pltpu.VMEM((C, D), jnp.float32),
        pltpu.VMEM((C, D), jnp.float32),
        pltpu.VMEM((ET,), jnp.float32),
        pltpu.VMEM((D,), jnp.float32),
        pltpu.VMEM((C,), jnp.float32),
        pltpu.VMEM_SHARED((N, D), jnp.float32),
        pltpu.VMEM_SHARED((N,), jnp.float32),
    ],
)


def _tc_pre_body(x_ref, w0_ref, b0_ref, w1_ref, b1_ref, f0_ref, f1_ref):
    x = x_ref[...]
    f0_ref[...] = (jnp.dot(x, w0_ref[...], preferred_element_type=jnp.float32)
                   + b0_ref[...])
    f1_ref[...] = (jnp.dot(x, w1_ref[...], preferred_element_type=jnp.float32)
                   + b1_ref[...])


_tc_pre = pl.pallas_call(
    _tc_pre_body,
    out_shape=[jax.ShapeDtypeStruct((N, D), jnp.float32),
               jax.ShapeDtypeStruct((N, D), jnp.float32)],
)


def _tc_mid_body(g0_ref, g1_ref, hp_ref, w0_ref, b0_ref, w1_ref, b1_ref,
                 h_ref, f0_ref, f1_ref):
    h = jax.nn.relu(g0_ref[...] + g1_ref[...] + 2.0 * hp_ref[...])
    h_ref[...] = h
    f0_ref[...] = (jnp.dot(h, w0_ref[...], preferred_element_type=jnp.float32)
                   + b0_ref[...])
    f1_ref[...] = (jnp.dot(h, w1_ref[...], preferred_element_type=jnp.float32)
                   + b1_ref[...])


_tc_mid = pl.pallas_call(
    _tc_mid_body,
    out_shape=[jax.ShapeDtypeStruct((N, D), jnp.float32),
               jax.ShapeDtypeStruct((N, D), jnp.float32),
               jax.ShapeDtypeStruct((N, D), jnp.float32)],
)


def _tc_post_body(g0_ref, g1_ref, hp_ref, o_ref):
    o_ref[...] = jax.nn.relu(g0_ref[...] + g1_ref[...] + 2.0 * hp_ref[...])


_tc_post = pl.pallas_call(
    _tc_post_body,
    out_shape=jax.ShapeDtypeStruct((N, D), jnp.float32),
)


def kernel(x, edge_index_r0, edge_index_r1,
           W_l0_r0, b_l0_r0, a_l0_r0, W_l0_r1, b_l0_r1, a_l0_r1,
           W_out_r0, b_out_r0, a_out_r0, W_out_r1, b_out_r1, a_out_r1):
    s0, d0 = edge_index_r0[0], edge_index_r0[1]
    s1, d1 = edge_index_r1[0], edge_index_r1[1]
    d20 = d0.reshape(NT, NCH, C)
    d21 = d1.reshape(NT, NCH, C)

    f00, f01 = _tc_pre(x, W_l0_r0, b_l0_r0, W_l0_r1, b_l0_r1)
    g00, g01 = _sc_gat(f00, f01, s0, s1, d20, d21, a_l0_r0, a_l0_r1)
    h, f10, f11 = _tc_mid(g00, g01, x, W_out_r0, b_out_r0, W_out_r1, b_out_r1)
    g10, g11 = _sc_gat(f10, f11, s0, s1, d20, d21, a_out_r0, a_out_r1)
    return _tc_post(g10, g11, h)


# speedup vs baseline: 8.7478x; 1.2689x over previous
import jax
import jax.numpy as jnp
from jax import lax
from jax.experimental import pallas as pl
from jax.experimental.pallas import tpu as pltpu
from jax.experimental.pallas import tpu_sc as plsc

N = 10000
E = 160000
D = 128
NT = 16
ET = E // NT
C = 80
NCH = ET // C
RPT = N // NT
NG = C // 16


def _run_relation(s, f, src_h, d2_h, a_h, out_h,
                  srcbuf, dst2d_v, rows_s, rows_d, exch, avmem,
                  acc, den):
    base = s * ET
    pltpu.sync_copy(d2_h.at[s], dst2d_v)
    pltpu.sync_copy(a_h, avmem)
    a_regs = [avmem[pl.ds(j * 16, 16)] for j in range(8)]

    lane15 = lax.iota(jnp.int32, 16) == 15

    def chunk1(i, _):
        off = i * C
        pltpu.sync_copy(src_h.at[pl.ds(base + off, C)], srcbuf.at[0])
        pltpu.sync_copy(f.at[srcbuf.at[0]], rows_s)
        pltpu.sync_copy(f.at[dst2d_v.at[i]], rows_d)

        def edge(e, _):
            acc_v = jnp.zeros((16,), jnp.float32)
            for j in range(8):
                sv = rows_s[e, pl.ds(j * 16, 16)]
                dv = rows_d[e, pl.ds(j * 16, 16)]
                t = sv + dv
                lr = jnp.maximum(t, 0.2 * t)
                acc_v = acc_v + lr * a_regs[j]
            cs = plsc.cumsum(acc_v)
            idxv = jnp.full((16,), e, jnp.int32)
            plsc.store_scatter(exch, [idxv], cs, mask=lane15)
            return 0
        lax.fori_loop(0, C, edge, 0)

        def grp(g, _):
            ex16 = jnp.exp(exch[pl.ds(g * 16, 16)])
            exch[pl.ds(g * 16, 16)] = ex16
            for e in range(16):
                a_s = ex16[e]
                ee = g * 16 + e
                for j in range(8):
                    rows_s[ee, pl.ds(j * 16, 16)] = (
                        rows_s[ee, pl.ds(j * 16, 16)] * a_s)
            return 0
        lax.fori_loop(0, NG, grp, 0)
        pltpu.sync_copy(rows_s, acc.at[dst2d_v.at[i]], add=True)
        pltpu.sync_copy(exch, den.at[dst2d_v.at[i]], add=True)
        return 0
    lax.fori_loop(0, NCH, chunk1, 0)

    plsc.subcore_barrier()

    def norm_block(rbase, nrows):
        pltpu.sync_copy(acc.at[pl.ds(rbase, nrows)],
                        rows_s.at[pl.ds(0, nrows)])
        pltpu.sync_copy(den.at[pl.ds(rbase, nrows)],
                        exch.at[pl.ds(0, nrows)])

        def grp(g, _):
            rec16 = 1.0 / jnp.maximum(exch[pl.ds(g * 16, 16)], 1e-9)
            for e in range(16):
                r_s = rec16[e]
                rr = g * 16 + e
                for j in range(8):
                    rows_s[rr, pl.ds(j * 16, 16)] = (
                        rows_s[rr, pl.ds(j * 16, 16)] * r_s)
            return 0
        lax.fori_loop(0, nrows // 16, grp, 0)
        pltpu.sync_copy(rows_s.at[pl.ds(0, nrows)],
                        out_h.at[pl.ds(rbase, nrows)])

    @pl.when(s < NT - 1)
    def _():
        rb = s * 624
        for k in range(7):
            norm_block(rb + k * 80, 80)
        norm_block(rb + 560, 64)

    @pl.when(s == NT - 1)
    def _():
        for k in range(8):
            norm_block(9360 + k * 80, 80)


def _sc_body(f0, f1, src0, src1, d20, d21, a0, a1,
             out0, out1,
             srcbuf, dst2d_v, rows_s, rows_d, exch, avmem, zv,
             acc, den):
    c = lax.axis_index("c")
    s = lax.axis_index("s")

    z16 = jnp.zeros((16,), jnp.float32)

    def zrow(r, _):
        for j in range(8):
            rows_s[r, pl.ds(j * 16, 16)] = z16
        return 0
    lax.fori_loop(0, C, zrow, 0)

    def zex(k, _):
        zv[pl.ds(k * 16, 16)] = z16
        return 0
    lax.fori_loop(0, 40, zex, 0)

    @pl.when(s < NT - 1)
    def _():
        base = s * 624
        for k in range(7):
            pltpu.sync_copy(rows_s, acc.at[pl.ds(base + k * 80, 80)])
        pltpu.sync_copy(rows_s.at[pl.ds(0, 64)], acc.at[pl.ds(base + 560, 64)])
        pltpu.sync_copy(zv, den.at[pl.ds(s * 640, 640)])

    @pl.when(s == NT - 1)
    def _():
        for k in range(8):
            pltpu.sync_copy(rows_s, acc.at[pl.ds(9360 + k * 80, 80)])
        pltpu.sync_copy(zv.at[pl.ds(0, 400)], den.at[pl.ds(9600, 400)])
    plsc.subcore_barrier()

    @pl.when(c == 0)
    def _():
        _run_relation(s, f0, src0, d20, a0, out0,
                      srcbuf, dst2d_v, rows_s, rows_d, exch, avmem,
                      acc, den)

    @pl.when(c == 1)
    def _():
        _run_relation(s, f1, src1, d21, a1, out1,
                      srcbuf, dst2d_v, rows_s, rows_d, exch, avmem,
                      acc, den)


_sc_gat = pl.kernel(
    _sc_body,
    out_type=[jax.ShapeDtypeStruct((N, D), jnp.float32),
              jax.ShapeDtypeStruct((N, D), jnp.float32)],
    mesh=plsc.VectorSubcoreMesh(core_axis_name="c", subcore_axis_name="s"),
    compiler_params=pltpu.CompilerParams(needs_layout_passes=False),
    scratch_types=[
        pltpu.VMEM((1, C), jnp.int32),
        pltpu.VMEM((NCH, C), jnp.int32),
        pltpu.VMEM((C, D), jnp.float32),
        pltpu.VMEM((C, D), jnp.float32),
        pltpu.VMEM((C,), jnp.float32),
        pltpu.VMEM((D,), jnp.float32),
        pltpu.VMEM((640,), jnp.float32),
        pltpu.VMEM_SHARED((N, D), jnp.float32),
        pltpu.VMEM_SHARED((N,), jnp.float32),
    ],
)


def _tc_pre_body(x_ref, w0_ref, b0_ref, w1_ref, b1_ref, f0_ref, f1_ref):
    x = x_ref[...]
    f0_ref[...] = (jnp.dot(x, w0_ref[...], preferred_element_type=jnp.float32)
                   + b0_ref[...])
    f1_ref[...] = (jnp.dot(x, w1_ref[...], preferred_element_type=jnp.float32)
                   + b1_ref[...])


_tc_pre = pl.pallas_call(
    _tc_pre_body,
    out_shape=[jax.ShapeDtypeStruct((N, D), jnp.float32),
               jax.ShapeDtypeStruct((N, D), jnp.float32)],
)


def _tc_mid_body(g0_ref, g1_ref, hp_ref, w0_ref, b0_ref, w1_ref, b1_ref,
                 h_ref, f0_ref, f1_ref):
    h = jax.nn.relu(g0_ref[...] + g1_ref[...] + 2.0 * hp_ref[...])
    h_ref[...] = h
    f0_ref[...] = (jnp.dot(h, w0_ref[...], preferred_element_type=jnp.float32)
                   + b0_ref[...])
    f1_ref[...] = (jnp.dot(h, w1_ref[...], preferred_element_type=jnp.float32)
                   + b1_ref[...])


_tc_mid = pl.pallas_call(
    _tc_mid_body,
    out_shape=[jax.ShapeDtypeStruct((N, D), jnp.float32),
               jax.ShapeDtypeStruct((N, D), jnp.float32),
               jax.ShapeDtypeStruct((N, D), jnp.float32)],
)


def _tc_post_body(g0_ref, g1_ref, hp_ref, o_ref):
    o_ref[...] = jax.nn.relu(g0_ref[...] + g1_ref[...] + 2.0 * hp_ref[...])


_tc_post = pl.pallas_call(
    _tc_post_body,
    out_shape=jax.ShapeDtypeStruct((N, D), jnp.float32),
)


def kernel(x, edge_index_r0, edge_index_r1,
           W_l0_r0, b_l0_r0, a_l0_r0, W_l0_r1, b_l0_r1, a_l0_r1,
           W_out_r0, b_out_r0, a_out_r0, W_out_r1, b_out_r1, a_out_r1):
    s0, d0 = edge_index_r0[0], edge_index_r0[1]
    s1, d1 = edge_index_r1[0], edge_index_r1[1]
    d20 = d0.reshape(NT, NCH, C)
    d21 = d1.reshape(NT, NCH, C)

    f00, f01 = _tc_pre(x, W_l0_r0, b_l0_r0, W_l0_r1, b_l0_r1)
    g00, g01 = _sc_gat(f00, f01, s0, s1, d20, d21, a_l0_r0, a_l0_r1)
    h, f10, f11 = _tc_mid(g00, g01, x, W_out_r0, b_out_r0, W_out_r1, b_out_r1)
    g10, g11 = _sc_gat(f10, f11, s0, s1, d20, d21, a_out_r0, a_out_r1)
    return _tc_post(g10, g11, h)


# speedup vs baseline: 15.8031x; 1.8065x over previous
import jax
import jax.numpy as jnp
from jax import lax
from jax.experimental import pallas as pl
from jax.experimental.pallas import tpu as pltpu
from jax.experimental.pallas import tpu_sc as plsc

N = 10000
E = 160000
D = 128
NT = 16
ET = E // NT
C = 80
NCH = ET // C
RPT = N // NT
NG = C // 16


def _run_relation(s, f, src_h, dst_h, a_h, out_h,
                  srcbuf, dstbuf, rows_s, rows_d, exch, avmem,
                  acc, den, sems):
    (gs, gd, sr, se, ixs, ixd) = sems
    base = s * ET
    pltpu.sync_copy(a_h, avmem)
    a_regs = [avmem[pl.ds(j * 16, 16)] for j in range(8)]
    lane15 = lax.iota(jnp.int32, 16) == 15

    def issue_idx(i, b, sl):
        off = base + i * C
        pltpu.async_copy(src_h.at[pl.ds(off, C)], srcbuf.at[b], ixs[b])
        pltpu.async_copy(dst_h.at[pl.ds(off, C)], dstbuf.at[sl], ixd[sl])

    def wait_idx(b, sl):
        pltpu.make_async_copy(src_h.at[pl.ds(0, C)], srcbuf.at[b],
                              ixs[b]).wait()
        pltpu.make_async_copy(dst_h.at[pl.ds(0, C)], dstbuf.at[sl],
                              ixd[sl]).wait()

    def issue_gather(b, sl):
        pltpu.async_copy(f.at[srcbuf.at[b]], rows_s.at[b], gs[b])
        pltpu.async_copy(f.at[dstbuf.at[sl]], rows_d.at[b], gd[b])

    def wait_gather(b, sl):
        pltpu.make_async_copy(f.at[srcbuf.at[b]], rows_s.at[b], gs[b]).wait()
        pltpu.make_async_copy(f.at[dstbuf.at[sl]], rows_d.at[b], gd[b]).wait()

    def issue_scatter(b, sl):
        pltpu.async_copy(rows_s.at[b], acc.at[dstbuf.at[sl]], sr[b], add=True)
        pltpu.async_copy(exch.at[b], den.at[dstbuf.at[sl]], se[b], add=True)

    def wait_scatter(b, sl):
        pltpu.make_async_copy(rows_s.at[b], acc.at[dstbuf.at[sl]],
                              sr[b]).wait()
        pltpu.make_async_copy(exch.at[b], den.at[dstbuf.at[sl]],
                              se[b]).wait()

    def compute(b):
        def edge(e, _):
            acc_v = jnp.zeros((16,), jnp.float32)
            for j in range(8):
                sv = rows_s[b, e, pl.ds(j * 16, 16)]
                dv = rows_d[b, e, pl.ds(j * 16, 16)]
                t = sv + dv
                lr = jnp.maximum(t, 0.2 * t)
                acc_v = acc_v + lr * a_regs[j]
            cs = plsc.cumsum(acc_v)
            idxv = jnp.full((16,), e, jnp.int32)
            plsc.store_scatter(exch.at[b], [idxv], cs, mask=lane15)
            return 0
        lax.fori_loop(0, C, edge, 0)

        def grp(g, _):
            ex16 = jnp.exp(exch[b, pl.ds(g * 16, 16)])
            exch[b, pl.ds(g * 16, 16)] = ex16
            for e in range(16):
                a_s = ex16[e]
                ee = g * 16 + e
                for j in range(8):
                    rows_s[b, ee, pl.ds(j * 16, 16)] = (
                        rows_s[b, ee, pl.ds(j * 16, 16)] * a_s)
            return 0
        lax.fori_loop(0, NG, grp, 0)

    def step(i, c):
        b, nb = c % 2, (c + 1) % 2
        sl, nsl, psl, isl = c, (c + 1) % 4, (c + 3) % 4, (c + 2) % 4
        wait_gather(b, sl)

        @pl.when(i + 2 < NCH)
        def _():
            issue_idx(i + 2, b, isl)

        @pl.when(i > 0)
        def _():
            wait_scatter(nb, psl)

        @pl.when(i + 1 < NCH)
        def _():
            wait_idx(nb, nsl)
            issue_gather(nb, nsl)
        compute(b)
        issue_scatter(b, sl)

    issue_idx(0, 0, 0)
    issue_idx(1, 1, 1)
    wait_idx(0, 0)
    issue_gather(0, 0)

    def group(k, _):
        i0 = k * 4
        step(i0, 0)
        step(i0 + 1, 1)
        step(i0 + 2, 2)
        step(i0 + 3, 3)
        return 0
    lax.fori_loop(0, NCH // 4, group, 0)
    step(NCH - 1, 0)
    wait_scatter(0, 0)

    plsc.subcore_barrier()

    def norm_block(rbase, nrows):
        pltpu.sync_copy(acc.at[pl.ds(rbase, nrows)],
                        rows_s.at[0, pl.ds(0, nrows)])
        pltpu.sync_copy(den.at[pl.ds(rbase, nrows)],
                        exch.at[0, pl.ds(0, nrows)])

        def grp(g, _):
            rec16 = 1.0 / jnp.maximum(exch[0, pl.ds(g * 16, 16)], 1e-9)
            for e in range(16):
                r_s = rec16[e]
                rr = g * 16 + e
                for j in range(8):
                    rows_s[0, rr, pl.ds(j * 16, 16)] = (
                        rows_s[0, rr, pl.ds(j * 16, 16)] * r_s)
            return 0
        lax.fori_loop(0, nrows // 16, grp, 0)
        pltpu.sync_copy(rows_s.at[0, pl.ds(0, nrows)],
                        out_h.at[pl.ds(rbase, nrows)])

    @pl.when(s < NT - 1)
    def _():
        rb = s * 624
        for k in range(7):
            norm_block(rb + k * 80, 80)
        norm_block(rb + 560, 64)

    @pl.when(s == NT - 1)
    def _():
        for k in range(8):
            norm_block(9360 + k * 80, 80)


def _sc_body(f0, f1, src0, src1, d10, d11, a0, a1,
             out0, out1,
             srcbuf, dstbuf, rows_s, rows_d, exch, avmem, zv,
             gs0, gs1, gd0, gd1, sr0, sr1, se0, se1, ixs0, ixs1,
             ixd0, ixd1, ixd2, ixd3,
             acc, den):
    c = lax.axis_index("c")
    s = lax.axis_index("s")
    sems = ((gs0, gs1), (gd0, gd1), (sr0, sr1), (se0, se1), (ixs0, ixs1),
            (ixd0, ixd1, ixd2, ixd3))

    z16 = jnp.zeros((16,), jnp.float32)

    def zrow(r, _):
        for j in range(8):
            rows_s[0, r, pl.ds(j * 16, 16)] = z16
        return 0
    lax.fori_loop(0, C, zrow, 0)

    def zex(k, _):
        zv[pl.ds(k * 16, 16)] = z16
        return 0
    lax.fori_loop(0, 40, zex, 0)

    @pl.when(s < NT - 1)
    def _():
        base = s * 624
        for k in range(7):
            pltpu.sync_copy(rows_s.at[0], acc.at[pl.ds(base + k * 80, 80)])
        pltpu.sync_copy(rows_s.at[0, pl.ds(0, 64)],
                        acc.at[pl.ds(base + 560, 64)])
        pltpu.sync_copy(zv, den.at[pl.ds(s * 640, 640)])

    @pl.when(s == NT - 1)
    def _():
        for k in range(8):
            pltpu.sync_copy(rows_s.at[0], acc.at[pl.ds(9360 + k * 80, 80)])
        pltpu.sync_copy(zv.at[pl.ds(0, 400)], den.at[pl.ds(9600, 400)])
    plsc.subcore_barrier()

    @pl.when(c == 0)
    def _():
        _run_relation(s, f0, src0, d10, a0, out0,
                      srcbuf, dstbuf, rows_s, rows_d, exch, avmem,
                      acc, den, sems)

    @pl.when(c == 1)
    def _():
        _run_relation(s, f1, src1, d11, a1, out1,
                      srcbuf, dstbuf, rows_s, rows_d, exch, avmem,
                      acc, den, sems)


_sc_gat = pl.kernel(
    _sc_body,
    out_type=[jax.ShapeDtypeStruct((N, D), jnp.float32),
              jax.ShapeDtypeStruct((N, D), jnp.float32)],
    mesh=plsc.VectorSubcoreMesh(core_axis_name="c", subcore_axis_name="s"),
    compiler_params=pltpu.CompilerParams(needs_layout_passes=False),
    scratch_types=[
        pltpu.VMEM((2, C), jnp.int32),
        pltpu.VMEM((4, C), jnp.int32),
        pltpu.VMEM((2, C, D), jnp.float32),
        pltpu.VMEM((2, C, D), jnp.float32),
        pltpu.VMEM((2, C), jnp.float32),
        pltpu.VMEM((D,), jnp.float32),
        pltpu.VMEM((640,), jnp.float32),
    ] + [pltpu.SemaphoreType.DMA] * 14 + [
        pltpu.VMEM_SHARED((N, D), jnp.float32),
        pltpu.VMEM_SHARED((N,), jnp.float32),
    ],
)


def _tc_pre_body(x_ref, w0_ref, b0_ref, w1_ref, b1_ref, f0_ref, f1_ref):
    x = x_ref[...]
    f0_ref[...] = (jnp.dot(x, w0_ref[...], preferred_element_type=jnp.float32)
                   + b0_ref[...])
    f1_ref[...] = (jnp.dot(x, w1_ref[...], preferred_element_type=jnp.float32)
                   + b1_ref[...])


_tc_pre = pl.pallas_call(
    _tc_pre_body,
    out_shape=[jax.ShapeDtypeStruct((N, D), jnp.float32),
               jax.ShapeDtypeStruct((N, D), jnp.float32)],
)


def _tc_mid_body(g0_ref, g1_ref, hp_ref, w0_ref, b0_ref, w1_ref, b1_ref,
                 h_ref, f0_ref, f1_ref):
    h = jax.nn.relu(g0_ref[...] + g1_ref[...] + 2.0 * hp_ref[...])
    h_ref[...] = h
    f0_ref[...] = (jnp.dot(h, w0_ref[...], preferred_element_type=jnp.float32)
                   + b0_ref[...])
    f1_ref[...] = (jnp.dot(h, w1_ref[...], preferred_element_type=jnp.float32)
                   + b1_ref[...])


_tc_mid = pl.pallas_call(
    _tc_mid_body,
    out_shape=[jax.ShapeDtypeStruct((N, D), jnp.float32),
               jax.ShapeDtypeStruct((N, D), jnp.float32),
               jax.ShapeDtypeStruct((N, D), jnp.float32)],
)


def _tc_post_body(g0_ref, g1_ref, hp_ref, o_ref):
    o_ref[...] = jax.nn.relu(g0_ref[...] + g1_ref[...] + 2.0 * hp_ref[...])


_tc_post = pl.pallas_call(
    _tc_post_body,
    out_shape=jax.ShapeDtypeStruct((N, D), jnp.float32),
)


def kernel(x, edge_index_r0, edge_index_r1,
           W_l0_r0, b_l0_r0, a_l0_r0, W_l0_r1, b_l0_r1, a_l0_r1,
           W_out_r0, b_out_r0, a_out_r0, W_out_r1, b_out_r1, a_out_r1):
    s0, d0 = edge_index_r0[0], edge_index_r0[1]
    s1, d1 = edge_index_r1[0], edge_index_r1[1]

    f00, f01 = _tc_pre(x, W_l0_r0, b_l0_r0, W_l0_r1, b_l0_r1)
    g00, g01 = _sc_gat(f00, f01, s0, s1, d0, d1, a_l0_r0, a_l0_r1)
    h, f10, f11 = _tc_mid(g00, g01, x, W_out_r0, b_out_r0, W_out_r1, b_out_r1)
    g10, g11 = _sc_gat(f10, f11, s0, s1, d0, d1, a_out_r0, a_out_r1)
    return _tc_post(g10, g11, h)


# speedup vs baseline: 25.1288x; 1.5901x over previous
import jax
import jax.numpy as jnp
from jax import lax
from jax.experimental import pallas as pl
from jax.experimental.pallas import tpu as pltpu
from jax.experimental.pallas import tpu_sc as plsc

N = 10000
E = 160000
D = 128
NT = 16
ET = E // NT
C = 80
NCH = ET // C
RPT = N // NT
NG = C // 16


def _run_relation(s, f, src_h, dst_h, a_h, out_h,
                  srcbuf, dstbuf, rows_s, rows_d, exch, avmem,
                  acc, den, sems):
    (gs, gd, sr, se, ixs, ixd) = sems
    base = s * ET
    pltpu.sync_copy(a_h, avmem)
    a_regs = [avmem[pl.ds(j * 16, 16)] for j in range(8)]
    lane15 = lax.iota(jnp.int32, 16) == 15

    def issue_idx(i, b, sl):
        off = base + i * C
        pltpu.async_copy(src_h.at[pl.ds(off, C)], srcbuf.at[b], ixs[b])
        pltpu.async_copy(dst_h.at[pl.ds(off, C)], dstbuf.at[sl], ixd[sl])

    def wait_idx(b, sl):
        pltpu.make_async_copy(src_h.at[pl.ds(0, C)], srcbuf.at[b],
                              ixs[b]).wait()
        pltpu.make_async_copy(dst_h.at[pl.ds(0, C)], dstbuf.at[sl],
                              ixd[sl]).wait()

    def issue_gather(b, sl):
        pltpu.async_copy(f.at[srcbuf.at[b]], rows_s.at[b], gs[b])
        pltpu.async_copy(f.at[dstbuf.at[sl]], rows_d.at[b], gd[b])

    def wait_gather(b, sl):
        pltpu.make_async_copy(f.at[srcbuf.at[b]], rows_s.at[b], gs[b]).wait()
        pltpu.make_async_copy(f.at[dstbuf.at[sl]], rows_d.at[b], gd[b]).wait()

    def issue_scatter(b, sl):
        pltpu.async_copy(rows_s.at[b], acc.at[dstbuf.at[sl]], sr[b], add=True)
        pltpu.async_copy(exch.at[b], den.at[dstbuf.at[sl]], se[b], add=True)

    def wait_scatter(b, sl):
        pltpu.make_async_copy(rows_s.at[b], acc.at[dstbuf.at[sl]],
                              sr[b]).wait()
        pltpu.make_async_copy(exch.at[b], den.at[dstbuf.at[sl]],
                              se[b]).wait()

    def compute(b):
        @plsc.parallel_loop(0, C, unroll=2)
        def _(e):
            svs = []
            acc_v = jnp.zeros((16,), jnp.float32)
            for j in range(8):
                sv = rows_s[b, e, pl.ds(j * 16, 16)]
                dv = rows_d[b, e, pl.ds(j * 16, 16)]
                svs.append(sv)
                t = sv + dv
                lr = jnp.maximum(t, 0.2 * t)
                acc_v = acc_v + lr * a_regs[j]
            cs = plsc.cumsum(acc_v)
            ex_v = jnp.exp(jnp.full((16,), cs[15], jnp.float32))
            idxv = jnp.full((16,), e, jnp.int32)
            plsc.store_scatter(exch.at[b], [idxv], ex_v, mask=lane15)
            for j in range(8):
                rows_s[b, e, pl.ds(j * 16, 16)] = svs[j] * ex_v

    def step(i, c):
        b, nb = c % 2, (c + 1) % 2
        sl, nsl, psl, isl = c, (c + 1) % 4, (c + 3) % 4, (c + 2) % 4
        wait_gather(b, sl)

        @pl.when(i + 2 < NCH)
        def _():
            issue_idx(i + 2, b, isl)

        @pl.when(i > 0)
        def _():
            wait_scatter(nb, psl)

        @pl.when(i + 1 < NCH)
        def _():
            wait_idx(nb, nsl)
            issue_gather(nb, nsl)
        compute(b)
        issue_scatter(b, sl)

    issue_idx(0, 0, 0)
    issue_idx(1, 1, 1)
    wait_idx(0, 0)
    issue_gather(0, 0)

    def group(k, _):
        i0 = k * 4
        step(i0, 0)
        step(i0 + 1, 1)
        step(i0 + 2, 2)
        step(i0 + 3, 3)
        return 0
    lax.fori_loop(0, NCH // 4, group, 0)
    step(NCH - 1, 0)
    wait_scatter(0, 0)

    plsc.subcore_barrier()

    def norm_block(rbase, nrows):
        pltpu.sync_copy(acc.at[pl.ds(rbase, nrows)],
                        rows_s.at[0, pl.ds(0, nrows)])
        pltpu.sync_copy(den.at[pl.ds(rbase, nrows)],
                        exch.at[0, pl.ds(0, nrows)])

        def grp(g, _):
            rec16 = 1.0 / jnp.maximum(exch[0, pl.ds(g * 16, 16)], 1e-9)
            for e in range(16):
                r_s = rec16[e]
                rr = g * 16 + e
                for j in range(8):
                    rows_s[0, rr, pl.ds(j * 16, 16)] = (
                        rows_s[0, rr, pl.ds(j * 16, 16)] * r_s)
            return 0
        lax.fori_loop(0, nrows // 16, grp, 0)
        pltpu.sync_copy(rows_s.at[0, pl.ds(0, nrows)],
                        out_h.at[pl.ds(rbase, nrows)])

    @pl.when(s < NT - 1)
    def _():
        rb = s * 624
        for k in range(7):
            norm_block(rb + k * 80, 80)
        norm_block(rb + 560, 64)

    @pl.when(s == NT - 1)
    def _():
        for k in range(8):
            norm_block(9360 + k * 80, 80)


def _sc_body(f0, f1, src0, src1, d10, d11, a0, a1,
             out0, out1,
             srcbuf, dstbuf, rows_s, rows_d, exch, avmem, zv,
             gs0, gs1, gd0, gd1, sr0, sr1, se0, se1, ixs0, ixs1,
             ixd0, ixd1, ixd2, ixd3,
             acc, den):
    c = lax.axis_index("c")
    s = lax.axis_index("s")
    sems = ((gs0, gs1), (gd0, gd1), (sr0, sr1), (se0, se1), (ixs0, ixs1),
            (ixd0, ixd1, ixd2, ixd3))

    z16 = jnp.zeros((16,), jnp.float32)

    def zrow(r, _):
        for j in range(8):
            rows_s[0, r, pl.ds(j * 16, 16)] = z16
        return 0
    lax.fori_loop(0, C, zrow, 0)

    def zex(k, _):
        zv[pl.ds(k * 16, 16)] = z16
        return 0
    lax.fori_loop(0, 40, zex, 0)

    @pl.when(s < NT - 1)
    def _():
        base = s * 624
        for k in range(7):
            pltpu.sync_copy(rows_s.at[0], acc.at[pl.ds(base + k * 80, 80)])
        pltpu.sync_copy(rows_s.at[0, pl.ds(0, 64)],
                        acc.at[pl.ds(base + 560, 64)])
        pltpu.sync_copy(zv, den.at[pl.ds(s * 640, 640)])

    @pl.when(s == NT - 1)
    def _():
        for k in range(8):
            pltpu.sync_copy(rows_s.at[0], acc.at[pl.ds(9360 + k * 80, 80)])
        pltpu.sync_copy(zv.at[pl.ds(0, 400)], den.at[pl.ds(9600, 400)])
    plsc.subcore_barrier()

    @pl.when(c == 0)
    def _():
        _run_relation(s, f0, src0, d10, a0, out0,
                      srcbuf, dstbuf, rows_s, rows_d, exch, avmem,
                      acc, den, sems)

    @pl.when(c == 1)
    def _():
        _run_relation(s, f1, src1, d11, a1, out1,
                      srcbuf, dstbuf, rows_s, rows_d, exch, avmem,
                      acc, den, sems)


_sc_gat = pl.kernel(
    _sc_body,
    out_type=[jax.ShapeDtypeStruct((N, D), jnp.float32),
              jax.ShapeDtypeStruct((N, D), jnp.float32)],
    mesh=plsc.VectorSubcoreMesh(core_axis_name="c", subcore_axis_name="s"),
    compiler_params=pltpu.CompilerParams(needs_layout_passes=False),
    scratch_types=[
        pltpu.VMEM((2, C), jnp.int32),
        pltpu.VMEM((4, C), jnp.int32),
        pltpu.VMEM((2, C, D), jnp.float32),
        pltpu.VMEM((2, C, D), jnp.float32),
        pltpu.VMEM((2, C), jnp.float32),
        pltpu.VMEM((D,), jnp.float32),
        pltpu.VMEM((640,), jnp.float32),
    ] + [pltpu.SemaphoreType.DMA] * 14 + [
        pltpu.VMEM_SHARED((N, D), jnp.float32),
        pltpu.VMEM_SHARED((N,), jnp.float32),
    ],
)


def _tc_pre_body(x_ref, w0_ref, b0_ref, w1_ref, b1_ref, f0_ref, f1_ref):
    x = x_ref[...]
    f0_ref[...] = (jnp.dot(x, w0_ref[...], preferred_element_type=jnp.float32)
                   + b0_ref[...])
    f1_ref[...] = (jnp.dot(x, w1_ref[...], preferred_element_type=jnp.float32)
                   + b1_ref[...])


_tc_pre = pl.pallas_call(
    _tc_pre_body,
    out_shape=[jax.ShapeDtypeStruct((N, D), jnp.float32),
               jax.ShapeDtypeStruct((N, D), jnp.float32)],
)


def _tc_mid_body(g0_ref, g1_ref, hp_ref, w0_ref, b0_ref, w1_ref, b1_ref,
                 h_ref, f0_ref, f1_ref):
    h = jax.nn.relu(g0_ref[...] + g1_ref[...] + 2.0 * hp_ref[...])
    h_ref[...] = h
    f0_ref[...] = (jnp.dot(h, w0_ref[...], preferred_element_type=jnp.float32)
                   + b0_ref[...])
    f1_ref[...] = (jnp.dot(h, w1_ref[...], preferred_element_type=jnp.float32)
                   + b1_ref[...])


_tc_mid = pl.pallas_call(
    _tc_mid_body,
    out_shape=[jax.ShapeDtypeStruct((N, D), jnp.float32),
               jax.ShapeDtypeStruct((N, D), jnp.float32),
               jax.ShapeDtypeStruct((N, D), jnp.float32)],
)


def _tc_post_body(g0_ref, g1_ref, hp_ref, o_ref):
    o_ref[...] = jax.nn.relu(g0_ref[...] + g1_ref[...] + 2.0 * hp_ref[...])


_tc_post = pl.pallas_call(
    _tc_post_body,
    out_shape=jax.ShapeDtypeStruct((N, D), jnp.float32),
)


def kernel(x, edge_index_r0, edge_index_r1,
           W_l0_r0, b_l0_r0, a_l0_r0, W_l0_r1, b_l0_r1, a_l0_r1,
           W_out_r0, b_out_r0, a_out_r0, W_out_r1, b_out_r1, a_out_r1):
    s0, d0 = edge_index_r0[0], edge_index_r0[1]
    s1, d1 = edge_index_r1[0], edge_index_r1[1]

    f00, f01 = _tc_pre(x, W_l0_r0, b_l0_r0, W_l0_r1, b_l0_r1)
    g00, g01 = _sc_gat(f00, f01, s0, s1, d0, d1, a_l0_r0, a_l0_r1)
    h, f10, f11 = _tc_mid(g00, g01, x, W_out_r0, b_out_r0, W_out_r1, b_out_r1)
    g10, g11 = _sc_gat(f10, f11, s0, s1, d0, d1, a_out_r0, a_out_r1)
    return _tc_post(g10, g11, h)


# speedup vs baseline: 25.4809x; 1.0140x over previous
import jax
import jax.numpy as jnp
from jax import lax
from jax.experimental import pallas as pl
from jax.experimental.pallas import tpu as pltpu
from jax.experimental.pallas import tpu_sc as plsc

N = 10000
E = 160000
D = 128
NT = 16
ET = E // NT
C = 80
NCH = ET // C
RPT = N // NT
NG = C // 16


def _run_relation(s, f, e2_h, a_h, out_h,
                  ibuf, rows_s, rows_d, exch, avmem,
                  acc, den, sems):
    (gs, gd, sr, se, ixd) = sems
    base = s * NCH
    pltpu.sync_copy(a_h, avmem)
    a_regs = [avmem[pl.ds(j * 16, 16)] for j in range(8)]
    lane15 = lax.iota(jnp.int32, 16) == 15

    def issue_idx(i, sl):
        pltpu.async_copy(e2_h.at[base + i], ibuf.at[sl], ixd[sl])

    def wait_idx(sl):
        pltpu.make_async_copy(e2_h.at[0], ibuf.at[sl], ixd[sl]).wait()

    def issue_gather(b, sl):
        pltpu.async_copy(f.at[ibuf.at[sl, 0]], rows_s.at[b], gs[b])
        pltpu.async_copy(f.at[ibuf.at[sl, 1]], rows_d.at[b], gd[b])

    def wait_gather(b, sl):
        pltpu.make_async_copy(f.at[ibuf.at[sl, 0]], rows_s.at[b],
                              gs[b]).wait()
        pltpu.make_async_copy(f.at[ibuf.at[sl, 1]], rows_d.at[b],
                              gd[b]).wait()

    def issue_scatter(b, sl):
        pltpu.async_copy(rows_s.at[b], acc.at[ibuf.at[sl, 1]], sr[b],
                         add=True)
        pltpu.async_copy(exch.at[b], den.at[ibuf.at[sl, 1]], se[b], add=True)

    def wait_scatter(b, sl):
        pltpu.make_async_copy(rows_s.at[b], acc.at[ibuf.at[sl, 1]],
                              sr[b]).wait()
        pltpu.make_async_copy(exch.at[b], den.at[ibuf.at[sl, 1]],
                              se[b]).wait()

    def compute(b):
        @plsc.parallel_loop(0, C, unroll=2)
        def _(e):
            svs = []
            acc_v = jnp.zeros((16,), jnp.float32)
            for j in range(8):
                sv = rows_s[b, e, pl.ds(j * 16, 16)]
                dv = rows_d[b, e, pl.ds(j * 16, 16)]
                svs.append(sv)
                t = sv + dv
                lr = jnp.maximum(t, 0.2 * t)
                acc_v = acc_v + lr * a_regs[j]
            cs = plsc.cumsum(acc_v)
            ex_v = jnp.exp(jnp.full((16,), cs[15], jnp.float32))
            idxv = jnp.full((16,), e, jnp.int32)
            plsc.store_scatter(exch.at[b], [idxv], ex_v, mask=lane15)
            for j in range(8):
                rows_s[b, e, pl.ds(j * 16, 16)] = svs[j] * ex_v

    def step(i, c):
        b, nb = c % 2, (c + 1) % 2
        sl, nsl, psl, isl = c, (c + 1) % 4, (c + 3) % 4, (c + 2) % 4
        wait_gather(b, sl)

        @pl.when(i + 2 < NCH)
        def _():
            issue_idx(i + 2, isl)

        @pl.when(i > 0)
        def _():
            wait_scatter(nb, psl)

        @pl.when(i + 1 < NCH)
        def _():
            wait_idx(nsl)
            issue_gather(nb, nsl)
        compute(b)
        issue_scatter(b, sl)

    issue_idx(0, 0)
    issue_idx(1, 1)
    wait_idx(0)
    issue_gather(0, 0)

    def group(k, _):
        i0 = k * 4
        step(i0, 0)
        step(i0 + 1, 1)
        step(i0 + 2, 2)
        step(i0 + 3, 3)
        return 0
    lax.fori_loop(0, NCH // 4, group, 0)
    step(NCH - 1, 0)
    wait_scatter(0, 0)

    plsc.subcore_barrier()

    def norm_pipeline(blocks):
        loads = {}

        def issue_load(k, b):
            rbase, nr = blocks[k]
            loads[k] = (
                pltpu.async_copy(acc.at[pl.ds(rbase, nr)],
                                 rows_s.at[b, pl.ds(0, nr)], gs[b]),
                pltpu.async_copy(den.at[pl.ds(rbase, nr)],
                                 exch.at[b, pl.ds(0, nr)], gd[b]),
            )

        issue_load(0, 0)
        store = {}
        for k in range(len(blocks)):
            b = k % 2
            rbase, nr = blocks[k]
            if k >= 1:
                store[k - 1].wait()
            if k + 1 < len(blocks):
                issue_load(k + 1, (k + 1) % 2)
            for hdl in loads[k]:
                hdl.wait()

            def grp(g, _, b=b):
                rec16 = 1.0 / jnp.maximum(exch[b, pl.ds(g * 16, 16)], 1e-9)
                for e in range(16):
                    r_s = rec16[e]
                    rr = g * 16 + e
                    for j in range(8):
                        rows_s[b, rr, pl.ds(j * 16, 16)] = (
                            rows_s[b, rr, pl.ds(j * 16, 16)] * r_s)
                return 0
            lax.fori_loop(0, nr // 16, grp, 0)
            store[k] = pltpu.async_copy(rows_s.at[b, pl.ds(0, nr)],
                                        out_h.at[pl.ds(rbase, nr)], sr[b])
        store[len(blocks) - 1].wait()

    @pl.when(s < NT - 1)
    def _():
        rb = s * 624
        norm_pipeline([(rb + k * 80, 80) for k in range(7)] + [(rb + 560, 64)])

    @pl.when(s == NT - 1)
    def _():
        norm_pipeline([(9360 + k * 80, 80) for k in range(8)])


def _sc_body(f0, f1, e20, e21, a0, a1,
             out0, out1,
             ibuf, rows_s, rows_d, exch, avmem, zv,
             gs0, gs1, gd0, gd1, sr0, sr1, se0, se1,
             ixd0, ixd1, ixd2, ixd3,
             acc, den):
    c = lax.axis_index("c")
    s = lax.axis_index("s")
    sems = ((gs0, gs1), (gd0, gd1), (sr0, sr1), (se0, se1),
            (ixd0, ixd1, ixd2, ixd3))

    z16 = jnp.zeros((16,), jnp.float32)

    def zrow(r, _):
        for j in range(8):
            rows_s[0, r, pl.ds(j * 16, 16)] = z16
        return 0
    lax.fori_loop(0, C, zrow, 0)

    def zex(k, _):
        zv[pl.ds(k * 16, 16)] = z16
        return 0
    lax.fori_loop(0, 40, zex, 0)

    hs = []

    @pl.when(s < NT - 1)
    def _():
        base = s * 624
        for k in range(7):
            hs.append(pltpu.async_copy(
                rows_s.at[0], acc.at[pl.ds(base + k * 80, 80)], gs0))
        hs.append(pltpu.async_copy(
            rows_s.at[0, pl.ds(0, 64)], acc.at[pl.ds(base + 560, 64)], gs0))
        hs.append(pltpu.async_copy(zv, den.at[pl.ds(s * 640, 640)], gd0))
        for hdl in hs:
            hdl.wait()

    @pl.when(s == NT - 1)
    def _():
        hs2 = []
        for k in range(8):
            hs2.append(pltpu.async_copy(
                rows_s.at[0], acc.at[pl.ds(9360 + k * 80, 80)], gs0))
        hs2.append(pltpu.async_copy(zv.at[pl.ds(0, 400)],
                                    den.at[pl.ds(9600, 400)], gd0))
        for hdl in hs2:
            hdl.wait()
    plsc.subcore_barrier()

    @pl.when(c == 0)
    def _():
        _run_relation(s, f0, e20, a0, out0,
                      ibuf, rows_s, rows_d, exch, avmem,
                      acc, den, sems)

    @pl.when(c == 1)
    def _():
        _run_relation(s, f1, e21, a1, out1,
                      ibuf, rows_s, rows_d, exch, avmem,
                      acc, den, sems)


_sc_gat = pl.kernel(
    _sc_body,
    out_type=[jax.ShapeDtypeStruct((N, D), jnp.float32),
              jax.ShapeDtypeStruct((N, D), jnp.float32)],
    mesh=plsc.VectorSubcoreMesh(core_axis_name="c", subcore_axis_name="s"),
    compiler_params=pltpu.CompilerParams(needs_layout_passes=False),
    scratch_types=[
        pltpu.VMEM((4, 2, C), jnp.int32),
        pltpu.VMEM((2, C, D), jnp.float32),
        pltpu.VMEM((2, C, D), jnp.float32),
        pltpu.VMEM((2, C), jnp.float32),
        pltpu.VMEM((D,), jnp.float32),
        pltpu.VMEM((640,), jnp.float32),
    ] + [pltpu.SemaphoreType.DMA] * 12 + [
        pltpu.VMEM_SHARED((N, D), jnp.float32),
        pltpu.VMEM_SHARED((N,), jnp.float32),
    ],
)


def _tc_pre_body(x_ref, w0_ref, b0_ref, w1_ref, b1_ref, f0_ref, f1_ref):
    x = x_ref[...]
    f0_ref[...] = (jnp.dot(x, w0_ref[...], preferred_element_type=jnp.float32)
                   + b0_ref[...])
    f1_ref[...] = (jnp.dot(x, w1_ref[...], preferred_element_type=jnp.float32)
                   + b1_ref[...])


_tc_pre = pl.pallas_call(
    _tc_pre_body,
    out_shape=[jax.ShapeDtypeStruct((N, D), jnp.float32),
               jax.ShapeDtypeStruct((N, D), jnp.float32)],
)


def _tc_mid_body(g0_ref, g1_ref, hp_ref, w0_ref, b0_ref, w1_ref, b1_ref,
                 h_ref, f0_ref, f1_ref):
    h = jax.nn.relu(g0_ref[...] + g1_ref[...] + 2.0 * hp_ref[...])
    h_ref[...] = h
    f0_ref[...] = (jnp.dot(h, w0_ref[...], preferred_element_type=jnp.float32)
                   + b0_ref[...])
    f1_ref[...] = (jnp.dot(h, w1_ref[...], preferred_element_type=jnp.float32)
                   + b1_ref[...])


_tc_mid = pl.pallas_call(
    _tc_mid_body,
    out_shape=[jax.ShapeDtypeStruct((N, D), jnp.float32),
               jax.ShapeDtypeStruct((N, D), jnp.float32),
               jax.ShapeDtypeStruct((N, D), jnp.float32)],
)


def _tc_post_body(g0_ref, g1_ref, hp_ref, o_ref):
    o_ref[...] = jax.nn.relu(g0_ref[...] + g1_ref[...] + 2.0 * hp_ref[...])


_tc_post = pl.pallas_call(
    _tc_post_body,
    out_shape=jax.ShapeDtypeStruct((N, D), jnp.float32),
)


def kernel(x, edge_index_r0, edge_index_r1,
           W_l0_r0, b_l0_r0, a_l0_r0, W_l0_r1, b_l0_r1, a_l0_r1,
           W_out_r0, b_out_r0, a_out_r0, W_out_r1, b_out_r1, a_out_r1):
    e20 = edge_index_r0.reshape(2, E // C, C).swapaxes(0, 1)
    e21 = edge_index_r1.reshape(2, E // C, C).swapaxes(0, 1)

    f00, f01 = _tc_pre(x, W_l0_r0, b_l0_r0, W_l0_r1, b_l0_r1)
    g00, g01 = _sc_gat(f00, f01, e20, e21, a_l0_r0, a_l0_r1)
    h, f10, f11 = _tc_mid(g00, g01, x, W_out_r0, b_out_r0, W_out_r1, b_out_r1)
    g10, g11 = _sc_gat(f10, f11, e20, e21, a_out_r0, a_out_r1)
    return _tc_post(g10, g11, h)
